# baseline (speedup 1.0000x reference)
import jax
import jax.numpy as jnp
from jax import lax
from jax.experimental import pallas as pl

BATCH = 16384
BLOCK = 4096
TEMP_INV = 1.0 / 5.0


def _mlp_block(xt_ref, a1_ref, a2_ref, o_ref):
    w1 = a1_ref[:, 0:20]
    b1 = a1_ref[:, 20:21]
    ht = lax.dot_general(w1, xt_ref[...], (((1,), (0,)), ((), ())),
                         preferred_element_type=jnp.float32)
    ht = jnp.maximum(ht + b1, 0.0)
    w2t = a2_ref[0:50, :]
    b2 = a2_ref[50:51, :]
    o = lax.dot_general(ht, w2t, (((0,), (0,)), ((), ())),
                        preferred_element_type=jnp.float32)
    o_ref[...] = (o + b2) * TEMP_INV


@jax.jit
def kernel(x, W1, b1, W2, b2):
    xt = x.T
    a1 = jnp.concatenate([W1, b1[:, None]], axis=1)
    a2 = jnp.concatenate([W2.T, b2[None, :]], axis=0)
    grid = (BATCH // BLOCK,)
    return pl.pallas_call(
        _mlp_block,
        grid=grid,
        in_specs=[
            pl.BlockSpec((xt.shape[0], BLOCK), lambda i: (0, i)),
            pl.BlockSpec(a1.shape, lambda i: (0, 0)),
            pl.BlockSpec(a2.shape, lambda i: (0, 0)),
        ],
        out_specs=pl.BlockSpec((BLOCK, W2.shape[0]), lambda i: (i, 0)),
        out_shape=jax.ShapeDtypeStruct((BATCH, W2.shape[0]), jnp.float32),
    )(xt, a1, a2)
